# Initial kernel scaffold; baseline (speedup 1.0000x reference)
import jax
import jax.numpy as jnp
from jax.experimental import pallas as pl


def kernel(logits):
    raise NotImplementedError("write your pallas kernel here")


# speedup vs baseline: 20.2596x; 20.2596x over previous
import functools

import jax
import jax.numpy as jnp
from jax import lax
from jax.experimental import pallas as pl
from jax.experimental.pallas import tpu as pltpu
from jax.experimental.pallas import tpu_sc as plsc

_A = 32 * 1024
_M = 128 * 512
_S = 32768
_TOTAL = _A + _M + _S

_NC = 2
_NS = 16
_NW = _NC * _NS
_CHUNK = _TOTAL // _NW
_A_W = _A // _CHUNK
_M_W = _M // _CHUNK


def _body(x_hbm, a_hbm, m_hbm, s_hbm, buf):
    wid = lax.axis_index("s") * _NC + lax.axis_index("c")
    src = wid * _CHUNK
    pltpu.sync_copy(x_hbm.at[pl.ds(src, _CHUNK)], buf)

    @pl.when(wid < _A_W)
    def _():
        pltpu.sync_copy(buf, a_hbm.at[pl.ds(src, _CHUNK)])

    @pl.when(jnp.logical_and(wid >= _A_W, wid < _A_W + _M_W))
    def _():
        pltpu.sync_copy(buf, m_hbm.at[pl.ds(src - _A, _CHUNK)])

    @pl.when(wid >= _A_W + _M_W)
    def _():
        pltpu.sync_copy(buf, s_hbm.at[pl.ds(src - _A - _M, _CHUNK)])


_scatter = functools.partial(
    pl.kernel,
    out_type=[
        jax.ShapeDtypeStruct((_A,), jnp.float32),
        jax.ShapeDtypeStruct((_M,), jnp.float32),
        jax.ShapeDtypeStruct((_S,), jnp.float32),
    ],
    mesh=plsc.VectorSubcoreMesh(core_axis_name="c", subcore_axis_name="s"),
    scratch_types=[pltpu.VMEM((_CHUNK,), jnp.float32)],
)(_body)


def kernel(logits):
    a, m, s = _scatter(logits)
    return {
        "attack": a.reshape(32, 1024),
        "move": m.reshape(128, 512),
        "select": s.reshape(1, 32768),
    }
